# baseline (speedup 1.0000x reference)
import jax
import jax.numpy as jnp
from jax.experimental import pallas as pl

K_SEL = 32
SLAB_W = 8
DBLK = 512


def _pair_stage(col, k, j, asc):
    n = len(col)
    for i in range(n):
        if i & j:
            continue
        p = i | j
        a, b = col[i], col[p]
        hi = jnp.maximum(a, b)
        lo = jnp.minimum(a, b)
        if ((i & k) == 0) != asc:
            col[i], col[p] = hi, lo
        else:
            col[i], col[p] = lo, hi


def _sort32(col, asc):
    n = len(col)
    k = 2
    while k < n:
        j = k // 2
        while j >= 1:
            _pair_stage(col, k, j, asc)
            j //= 2
        k *= 2
    j = n // 2
    while j >= 1:
        _pair_stage(col, n, j, asc)
        j //= 2


def _merge_net(col, asc):
    n = len(col)
    j = n // 2
    while j >= 1:
        _pair_stage(col, n, j, asc)
        j //= 2


def _topk_kernel(x_ref, o_ref):
    n = K_SEL
    cols = x_ref.shape[2]
    nslab = cols // SLAB_W
    entries = []
    for s in range(nslab):
        col = [x_ref[0, i, s * SLAB_W:(s + 1) * SLAB_W, :] for i in range(n)]
        _sort32(col, asc=s >= nslab // 2)
        entries.append(col)
    width = SLAB_W
    while len(entries) > 1 or width > 1:
        half = len(entries) // 2
        entries = [[jnp.maximum(a, b)
                    for a, b in zip(entries[t], entries[half + t])]
                   for t in range(half)]
        if len(entries) == 1 and width > 1:
            h = width // 2
            top = [a[:h] for a in entries[0]]
            bot = [a[h:] for a in entries[0]]
            entries = [top, bot]
            width = h
        done = len(entries) == 1 and width == 1
        ne = len(entries)
        for t, e in enumerate(entries):
            _merge_net(e, asc=(not done) and t >= ne // 2)
    o_ref[0] = jnp.concatenate(entries[0], axis=0)


@jax.jit
def kernel(x):
    b, l, d = x.shape
    k = K_SEL
    cols = l // k
    xr = x.reshape(b, k, cols, d)
    dblk = DBLK
    grid = (b, d // dblk)
    out = pl.pallas_call(
        _topk_kernel,
        grid=grid,
        in_specs=[
            pl.BlockSpec((1, k, cols, dblk), lambda i, j: (i, 0, 0, j)),
        ],
        out_specs=pl.BlockSpec((1, k, dblk), lambda i, j: (i, 0, j)),
        out_shape=jax.ShapeDtypeStruct((b, k, d), x.dtype),
    )(xr)
    return jnp.swapaxes(out, 1, 2).reshape(b, d * k)


# speedup vs baseline: 1.0085x; 1.0085x over previous
import jax
import jax.numpy as jnp
from jax.experimental import pallas as pl

K_SEL = 32
SLAB_W = 8
DBLK = 256


def _pair_stage(col, k, j, asc):
    n = len(col)
    for i in range(n):
        if i & j:
            continue
        p = i | j
        a, b = col[i], col[p]
        hi = jnp.maximum(a, b)
        lo = jnp.minimum(a, b)
        if ((i & k) == 0) != asc:
            col[i], col[p] = hi, lo
        else:
            col[i], col[p] = lo, hi


def _sort32(col, asc):
    n = len(col)
    k = 2
    while k < n:
        j = k // 2
        while j >= 1:
            _pair_stage(col, k, j, asc)
            j //= 2
        k *= 2
    j = n // 2
    while j >= 1:
        _pair_stage(col, n, j, asc)
        j //= 2


def _merge_net(col, asc):
    n = len(col)
    j = n // 2
    while j >= 1:
        _pair_stage(col, n, j, asc)
        j //= 2


def _topk_kernel(x_ref, o_ref):
    n = K_SEL
    cols = x_ref.shape[2]
    nslab = cols // SLAB_W
    entries = []
    for s in range(nslab):
        col = [x_ref[0, i, s * SLAB_W:(s + 1) * SLAB_W, :] for i in range(n)]
        _sort32(col, asc=s >= nslab // 2)
        entries.append(col)
    width = SLAB_W
    while len(entries) > 1 or width > 1:
        half = len(entries) // 2
        entries = [[jnp.maximum(a, b)
                    for a, b in zip(entries[t], entries[half + t])]
                   for t in range(half)]
        if len(entries) == 1 and width > 1:
            h = width // 2
            top = [a[:h] for a in entries[0]]
            bot = [a[h:] for a in entries[0]]
            entries = [top, bot]
            width = h
        done = len(entries) == 1 and width == 1
        ne = len(entries)
        for t, e in enumerate(entries):
            _merge_net(e, asc=(not done) and t >= ne // 2)
    o_ref[0] = jnp.concatenate(entries[0], axis=0)


@jax.jit
def kernel(x):
    b, l, d = x.shape
    k = K_SEL
    cols = l // k
    xr = x.reshape(b, k, cols, d)
    dblk = DBLK
    grid = (b, d // dblk)
    out = pl.pallas_call(
        _topk_kernel,
        grid=grid,
        in_specs=[
            pl.BlockSpec((1, k, cols, dblk), lambda i, j: (i, 0, 0, j)),
        ],
        out_specs=pl.BlockSpec((1, k, dblk), lambda i, j: (i, 0, j)),
        out_shape=jax.ShapeDtypeStruct((b, k, d), x.dtype),
    )(xr)
    return jnp.swapaxes(out, 1, 2).reshape(b, d * k)
